# baseline (speedup 1.0000x reference)
import dataclasses
import functools

import jax
import jax.numpy as jnp
import numpy as np
from jax import lax
from jax.experimental import pallas as pl
from jax.experimental.pallas import tpu as pltpu
from jax.experimental.pallas import tpu_sc as plsc

N = 10000
E = 320000
D = 128
A = 10
RB = 8
H = 64
AVG = 32.0

NC = 2
NS = 16
NW = NC * NS

S = 1
E_SL = E // S
EPW = E_SL // NW
C = 80
NCHUNK = EPW // C
NR = 624

EB = 3200


def _silu(v):
    return v * (0.5 + 0.5 * jnp.tanh(0.5 * v))


def _x_up(node_feats, W_up):
    nb = 2000

    def body(nf_ref, w_ref, o_ref):
        o_ref[...] = jnp.dot(nf_ref[...], w_ref[...],
                             preferred_element_type=jnp.float32)

    return pl.pallas_call(
        body,
        grid=(N // nb,),
        in_specs=[pl.BlockSpec((nb, D), lambda i: (i, 0)),
                  pl.BlockSpec((D, D), lambda i: (0, 0))],
        out_specs=pl.BlockSpec((nb, D), lambda i: (i, 0)),
        out_shape=jax.ShapeDtypeStruct((N, D), jnp.float32),
    )(node_feats, W_up)


def _edge_weights(ef_t, ea_t, R1, R2, R3, R4, s):
    bf = jnp.bfloat16
    blk0 = s * (E_SL // EB)
    dn_t = (((0,), (0,)), ((), ()))

    def body(ef_ref, ea_ref, r1, r2, r3, r4, o_ref):
        h = _silu(lax.dot_general(ef_ref[...], r1[...], dn_t,
                                  preferred_element_type=jnp.float32))
        h = _silu(jnp.dot(h.astype(bf), r2[...],
                          preferred_element_type=jnp.float32))
        h = _silu(jnp.dot(h.astype(bf), r3[...],
                          preferred_element_type=jnp.float32))
        ea_col = lax.dot_general(ea_ref[...], jnp.ones((1, 1), jnp.float32),
                                 dn_t, preferred_element_type=jnp.float32)
        o_ref[...] = (jnp.dot(h.astype(bf), r4[...],
                              preferred_element_type=jnp.float32)
                      * ea_col).astype(bf)

    return pl.pallas_call(
        body,
        grid=(E_SL // EB,),
        in_specs=[pl.BlockSpec((RB, EB), lambda i: (0, i + blk0)),
                  pl.BlockSpec((1, EB), lambda i: (0, i + blk0)),
                  pl.BlockSpec((RB, H), lambda i: (0, 0)),
                  pl.BlockSpec((H, H), lambda i: (0, 0)),
                  pl.BlockSpec((H, H), lambda i: (0, 0)),
                  pl.BlockSpec((H, D), lambda i: (0, 0))],
        out_specs=pl.BlockSpec((EB, D), lambda i: (i, 0)),
        out_shape=jax.ShapeDtypeStruct((E_SL, D), jnp.bfloat16),
    )(ef_t, ea_t, R1, R2, R3, R4)


def _sc_message(x, w, ei_flat, eoff0):
    mesh = plsc.VectorSubcoreMesh(core_axis_name="c", subcore_axis_name="s")

    @functools.partial(
        pl.kernel,
        out_type=jax.ShapeDtypeStruct((NC, N, D), jnp.float32),
        mesh=mesh,
        scratch_types=[
            pltpu.VMEM((4, C), jnp.int32),
            pltpu.VMEM((4, C), jnp.int32),
            pltpu.VMEM((2, C, D), jnp.float32),
            pltpu.VMEM((2, C, D), jnp.bfloat16),
            pltpu.VMEM_SHARED((N, D), jnp.float32),
            pltpu.SemaphoreType.DMA((4,)),
            pltpu.SemaphoreType.DMA((2,)),
            pltpu.SemaphoreType.DMA((2,)),
            pltpu.SemaphoreType.DMA((2,)),
        ],
    )
    def k(x_hbm, w_hbm, ei_hbm, out_hbm,
          si_v, ri_v, xs_v, wv_v, acc_sh, sem_i, sem_g, sem_w, sem_s):
        cid = lax.axis_index("c")
        sid = lax.axis_index("s")
        base = eoff0 + (cid * NS + sid) * EPW
        wbase = (cid * NS + sid) * EPW

        def issue_idx(g, slot):
            eoff = base + g * C
            pltpu.async_copy(ei_hbm.at[pl.ds(eoff, C)], si_v.at[slot],
                             sem_i.at[slot])
            pltpu.async_copy(ei_hbm.at[pl.ds(E + eoff, C)], ri_v.at[slot],
                             sem_i.at[slot])

        def wait_idx(g, slot):
            eoff = base + g * C
            pltpu.make_async_copy(ei_hbm.at[pl.ds(eoff, C)], si_v.at[slot],
                                  sem_i.at[slot]).wait()
            pltpu.make_async_copy(ei_hbm.at[pl.ds(E + eoff, C)],
                                  ri_v.at[slot], sem_i.at[slot]).wait()

        def issue_gather(slot, p):
            pltpu.async_copy(x_hbm.at[si_v.at[slot]], xs_v.at[p], sem_g.at[p])

        def wait_gather(slot, p):
            pltpu.make_async_copy(x_hbm.at[si_v.at[slot]], xs_v.at[p],
                                  sem_g.at[p]).wait()

        def issue_w(g, p):
            eoff = wbase + g * C
            pltpu.async_copy(w_hbm.at[pl.ds(eoff, C)], wv_v.at[p],
                             sem_w.at[p])

        def wait_w(g, p):
            eoff = wbase + g * C
            pltpu.make_async_copy(w_hbm.at[pl.ds(eoff, C)], wv_v.at[p],
                                  sem_w.at[p]).wait()

        def issue_scatter(p, slot):
            pltpu.async_copy(xs_v.at[p], acc_sh.at[ri_v.at[slot]],
                             sem_s.at[p], add=True)

        def wait_scatter(p, slot):
            pltpu.make_async_copy(xs_v.at[p], acc_sh.at[ri_v.at[slot]],
                                  sem_s.at[p]).wait()

        def multiply(p):
            xs_p = xs_v.at[p]
            w_p = wv_v.at[p]

            @pl.loop(0, C // 2)
            def _(t):
                i2 = pl.multiple_of(2 * t, 2)
                for j in range(D // 16):
                    sl = (pl.ds(i2, 2), pl.ds(j * 16, 16))
                    xs_p.at[sl][...] = (
                        xs_p.at[sl][...]
                        * w_p.at[sl][...].astype(jnp.float32))

        @pl.loop(0, C)
        def _(i):
            for j in range(D // 16):
                xs_v.at[0].at[pl.ds(i, 1), pl.ds(j * 16, 16)][...] = (
                    jnp.zeros((1, 16), jnp.float32))

        row0 = sid * NR
        zchunks = [C] * (NR // C)
        if NR % C:
            zchunks.append(NR % C)
        off = 0
        for cnt in zchunks:
            pltpu.sync_copy(xs_v.at[0].at[pl.ds(0, cnt)],
                            acc_sh.at[pl.ds(row0 + off, cnt)])
            off += cnt

        @pl.when(sid == NS - 1)
        def _():
            pltpu.sync_copy(xs_v.at[0].at[pl.ds(0, N - NS * NR)],
                            acc_sh.at[pl.ds(NS * NR, N - NS * NR)])

        plsc.subcore_barrier()

        issue_idx(0, 0)
        issue_idx(1, 1)
        wait_idx(0, 0)
        issue_gather(0, 0)
        issue_w(0, 0)

        @pl.loop(0, (NCHUNK - 1) // 4)
        def _(t):
            for b in range(4):
                g = t * 4 + b
                s1 = (b + 1) % 4
                p = b % 2
                q = (b + 1) % 2
                wait_idx(g + 1, s1)

                @pl.when(g >= 1)
                def _():
                    wait_scatter(q, (b + 3) % 4)

                issue_gather(s1, q)
                issue_w(g + 1, q)

                @pl.when(g + 2 <= NCHUNK - 1)
                def _():
                    issue_idx(g + 2, (b + 2) % 4)

                wait_gather(b % 4, p)
                wait_w(g, p)
                multiply(p)
                issue_scatter(p, b % 4)

        wait_scatter(1, 3)
        wait_gather(0, 0)
        wait_w(NCHUNK - 1, 0)
        multiply(0)
        pltpu.sync_copy(xs_v.at[0], acc_sh.at[ri_v.at[0]], add=True)

        plsc.subcore_barrier()
        pltpu.sync_copy(acc_sh.at[pl.ds(row0, NR)],
                        out_hbm.at[cid, pl.ds(row0, NR)])

        @pl.when(sid == NS - 1)
        def _():
            pltpu.sync_copy(acc_sh.at[pl.ds(NS * NR, N - NS * NR)],
                            out_hbm.at[cid, pl.ds(NS * NR, N - NS * NR)])

    return k(x, w, ei_flat)


def _readout(msgs, node_attrs, W_lin, W_skip):
    nb = 2000
    nm = len(msgs)

    def body(*refs):
        msg_refs = refs[:nm]
        na_ref, wl_ref, ws_ref, o_ref = refs[nm:]
        m = msg_refs[0][...]
        for r in msg_refs[1:]:
            m = m + r[...]
        m2 = jnp.dot(m, wl_ref[...],
                     preferred_element_type=jnp.float32) * (1.0 / AVG)
        na = na_ref[...]
        ws = ws_ref[...]
        acc = jnp.dot(m2 * na[:, 0:1], ws[:, 0, :],
                      preferred_element_type=jnp.float32)
        for a in range(1, A):
            acc = acc + jnp.dot(m2 * na[:, a:a + 1], ws[:, a, :],
                                preferred_element_type=jnp.float32)
        o_ref[...] = acc

    return pl.pallas_call(
        body,
        grid=(N // nb,),
        in_specs=[pl.BlockSpec((nb, D), lambda i: (i, 0))] * nm + [
            pl.BlockSpec((nb, A), lambda i: (i, 0)),
            pl.BlockSpec((D, D), lambda i: (0, 0)),
            pl.BlockSpec((D, A, D), lambda i: (0, 0, 0))],
        out_specs=pl.BlockSpec((nb, D), lambda i: (i, 0)),
        out_shape=jax.ShapeDtypeStruct((N, D), jnp.float32),
    )(*msgs, node_attrs, W_lin, W_skip)


def kernel(node_attrs, node_feats, edge_attrs, edge_feats, edge_index,
           W_up, R1, R2, R3, R4, W_lin, W_skip):
    bf = jnp.bfloat16
    ei_flat = edge_index.reshape(2 * E)
    ef_t = edge_feats.T.astype(bf)
    ea_t = edge_attrs.reshape(1, E)
    r1, r2, r3, r4 = (R1.astype(bf), R2.astype(bf), R3.astype(bf),
                      R4.astype(bf))
    x = _x_up(node_feats, W_up)
    msgs = []
    for s in range(S):
        w_s = _edge_weights(ef_t, ea_t, r1, r2, r3, r4, s)
        parts = _sc_message(x, w_s, ei_flat, s * E_SL)
        msgs.append(parts[0])
        msgs.append(parts[1])
    return _readout(msgs, node_attrs, W_lin, W_skip)


# speedup vs baseline: 1.3283x; 1.3283x over previous
import dataclasses
import functools

import jax
import jax.numpy as jnp
import numpy as np
from jax import lax
from jax.experimental import pallas as pl
from jax.experimental.pallas import tpu as pltpu
from jax.experimental.pallas import tpu_sc as plsc

N = 10000
E = 320000
D = 128
A = 10
RB = 8
H = 64
AVG = 32.0

NC = 2
NS = 16
NW = NC * NS

S = 2
E_SL = E // S
EPW = E_SL // NW
C = 40
NCHUNK = EPW // C
NR = 624

EB = 3200


def _silu(v):
    return v * (0.5 + 0.5 * jnp.tanh(0.5 * v))


def _x_up(node_feats, W_up):
    nb = 2000

    def body(nf_ref, w_ref, o_ref):
        o_ref[...] = jnp.dot(nf_ref[...], w_ref[...],
                             preferred_element_type=jnp.float32)

    return pl.pallas_call(
        body,
        grid=(N // nb,),
        in_specs=[pl.BlockSpec((nb, D), lambda i: (i, 0)),
                  pl.BlockSpec((D, D), lambda i: (0, 0))],
        out_specs=pl.BlockSpec((nb, D), lambda i: (i, 0)),
        out_shape=jax.ShapeDtypeStruct((N, D), jnp.float32),
    )(node_feats, W_up)


def _edge_weights(ef_t, ea_t, R1, R2, R3, R4, s):
    bf = jnp.bfloat16
    blk0 = s * (E_SL // EB)
    dn_t = (((0,), (0,)), ((), ()))

    def body(ef_ref, ea_ref, r1, r2, r3, r4, o_ref):
        h = _silu(lax.dot_general(ef_ref[...], r1[...], dn_t,
                                  preferred_element_type=jnp.float32))
        h = _silu(jnp.dot(h.astype(bf), r2[...],
                          preferred_element_type=jnp.float32))
        h = _silu(jnp.dot(h.astype(bf), r3[...],
                          preferred_element_type=jnp.float32))
        ea_col = lax.dot_general(ea_ref[...], jnp.ones((1, 1), jnp.float32),
                                 dn_t, preferred_element_type=jnp.float32)
        o_ref[...] = jnp.dot(h.astype(bf), r4[...],
                             preferred_element_type=jnp.float32) * ea_col

    return pl.pallas_call(
        body,
        grid=(E_SL // EB,),
        in_specs=[pl.BlockSpec((RB, EB), lambda i: (0, i + blk0)),
                  pl.BlockSpec((1, EB), lambda i: (0, i + blk0)),
                  pl.BlockSpec((RB, H), lambda i: (0, 0)),
                  pl.BlockSpec((H, H), lambda i: (0, 0)),
                  pl.BlockSpec((H, H), lambda i: (0, 0)),
                  pl.BlockSpec((H, D), lambda i: (0, 0))],
        out_specs=pl.BlockSpec((EB, D), lambda i: (i, 0)),
        out_shape=jax.ShapeDtypeStruct((E_SL, D), jnp.float32),
    )(ef_t, ea_t, R1, R2, R3, R4)


def _sc_message(x, w, ei_flat, eoff0):
    mesh = plsc.VectorSubcoreMesh(core_axis_name="c", subcore_axis_name="s")

    @functools.partial(
        pl.kernel,
        out_type=jax.ShapeDtypeStruct((NC, N, D), jnp.float32),
        mesh=mesh,
        scratch_types=[
            pltpu.VMEM((4, C), jnp.int32),
            pltpu.VMEM((4, C), jnp.int32),
            pltpu.VMEM((2, C, D), jnp.float32),
            pltpu.VMEM((2, C, D), jnp.float32),
            pltpu.VMEM_SHARED((N, D), jnp.float32),
            pltpu.SemaphoreType.DMA((4,)),
            pltpu.SemaphoreType.DMA((2,)),
            pltpu.SemaphoreType.DMA((2,)),
            pltpu.SemaphoreType.DMA((2,)),
        ],
    )
    def k(x_hbm, w_hbm, ei_hbm, out_hbm,
          si_v, ri_v, xs_v, wv_v, acc_sh, sem_i, sem_g, sem_w, sem_s):
        cid = lax.axis_index("c")
        sid = lax.axis_index("s")
        base = eoff0 + (cid * NS + sid) * EPW
        wbase = (cid * NS + sid) * EPW

        def issue_idx(g, slot):
            eoff = base + g * C
            pltpu.async_copy(ei_hbm.at[pl.ds(eoff, C)], si_v.at[slot],
                             sem_i.at[slot])
            pltpu.async_copy(ei_hbm.at[pl.ds(E + eoff, C)], ri_v.at[slot],
                             sem_i.at[slot])

        def wait_idx(g, slot):
            eoff = base + g * C
            pltpu.make_async_copy(ei_hbm.at[pl.ds(eoff, C)], si_v.at[slot],
                                  sem_i.at[slot]).wait()
            pltpu.make_async_copy(ei_hbm.at[pl.ds(E + eoff, C)],
                                  ri_v.at[slot], sem_i.at[slot]).wait()

        def issue_gather(slot, p):
            pltpu.async_copy(x_hbm.at[si_v.at[slot]], xs_v.at[p], sem_g.at[p])

        def wait_gather(slot, p):
            pltpu.make_async_copy(x_hbm.at[si_v.at[slot]], xs_v.at[p],
                                  sem_g.at[p]).wait()

        def issue_w(g, p):
            eoff = wbase + g * C
            pltpu.async_copy(w_hbm.at[pl.ds(eoff, C)], wv_v.at[p],
                             sem_w.at[p])

        def wait_w(g, p):
            eoff = wbase + g * C
            pltpu.make_async_copy(w_hbm.at[pl.ds(eoff, C)], wv_v.at[p],
                                  sem_w.at[p]).wait()

        def issue_scatter(p, slot):
            pltpu.async_copy(xs_v.at[p], acc_sh.at[ri_v.at[slot]],
                             sem_s.at[p], add=True)

        def wait_scatter(p, slot):
            pltpu.make_async_copy(xs_v.at[p], acc_sh.at[ri_v.at[slot]],
                                  sem_s.at[p]).wait()

        def multiply(p):
            xs_p = xs_v.at[p]
            w_p = wv_v.at[p]

            @pl.loop(0, C)
            def _(i):
                for j in range(D // 16):
                    sl = (pl.ds(i, 1), pl.ds(j * 16, 16))
                    xs_p.at[sl][...] = xs_p.at[sl][...] * w_p.at[sl][...]

        @pl.loop(0, C)
        def _(i):
            for j in range(D // 16):
                xs_v.at[0].at[pl.ds(i, 1), pl.ds(j * 16, 16)][...] = (
                    jnp.zeros((1, 16), jnp.float32))

        row0 = sid * NR
        zchunks = [C] * (NR // C)
        if NR % C:
            zchunks.append(NR % C)
        off = 0
        for cnt in zchunks:
            pltpu.sync_copy(xs_v.at[0].at[pl.ds(0, cnt)],
                            acc_sh.at[pl.ds(row0 + off, cnt)])
            off += cnt

        @pl.when(sid == NS - 1)
        def _():
            pltpu.sync_copy(xs_v.at[0].at[pl.ds(0, N - NS * NR)],
                            acc_sh.at[pl.ds(NS * NR, N - NS * NR)])

        plsc.subcore_barrier()

        issue_idx(0, 0)
        issue_idx(1, 1)
        wait_idx(0, 0)
        issue_gather(0, 0)
        issue_w(0, 0)

        @pl.loop(0, (NCHUNK - 1) // 4)
        def _(t):
            for b in range(4):
                g = t * 4 + b
                s1 = (b + 1) % 4
                p = b % 2
                q = (b + 1) % 2
                wait_idx(g + 1, s1)

                @pl.when(g >= 1)
                def _():
                    wait_scatter(q, (b + 3) % 4)

                issue_gather(s1, q)
                issue_w(g + 1, q)

                @pl.when(g + 2 <= NCHUNK - 1)
                def _():
                    issue_idx(g + 2, (b + 2) % 4)

                wait_gather(b % 4, p)
                wait_w(g, p)
                multiply(p)
                issue_scatter(p, b % 4)

        wait_scatter(1, 3)
        wait_gather(0, 0)
        wait_w(NCHUNK - 1, 0)
        multiply(0)
        pltpu.sync_copy(xs_v.at[0], acc_sh.at[ri_v.at[0]], add=True)

        plsc.subcore_barrier()
        pltpu.sync_copy(acc_sh.at[pl.ds(row0, NR)],
                        out_hbm.at[cid, pl.ds(row0, NR)])

        @pl.when(sid == NS - 1)
        def _():
            pltpu.sync_copy(acc_sh.at[pl.ds(NS * NR, N - NS * NR)],
                            out_hbm.at[cid, pl.ds(NS * NR, N - NS * NR)])

    return k(x, w, ei_flat)


def _readout(msgs, node_attrs, W_lin, W_skip):
    nb = 2000
    nm = len(msgs)

    def body(*refs):
        msg_refs = refs[:nm]
        na_ref, wl_ref, ws_ref, o_ref = refs[nm:]
        m = msg_refs[0][...]
        for r in msg_refs[1:]:
            m = m + r[...]
        m2 = jnp.dot(m, wl_ref[...],
                     preferred_element_type=jnp.float32) * (1.0 / AVG)
        na = na_ref[...]
        ws = ws_ref[...]
        acc = jnp.dot(m2 * na[:, 0:1], ws[:, 0, :],
                      preferred_element_type=jnp.float32)
        for a in range(1, A):
            acc = acc + jnp.dot(m2 * na[:, a:a + 1], ws[:, a, :],
                                preferred_element_type=jnp.float32)
        o_ref[...] = acc

    return pl.pallas_call(
        body,
        grid=(N // nb,),
        in_specs=[pl.BlockSpec((nb, D), lambda i: (i, 0))] * nm + [
            pl.BlockSpec((nb, A), lambda i: (i, 0)),
            pl.BlockSpec((D, D), lambda i: (0, 0)),
            pl.BlockSpec((D, A, D), lambda i: (0, 0, 0))],
        out_specs=pl.BlockSpec((nb, D), lambda i: (i, 0)),
        out_shape=jax.ShapeDtypeStruct((N, D), jnp.float32),
    )(*msgs, node_attrs, W_lin, W_skip)


def kernel(node_attrs, node_feats, edge_attrs, edge_feats, edge_index,
           W_up, R1, R2, R3, R4, W_lin, W_skip):
    bf = jnp.bfloat16
    ei_flat = edge_index.reshape(2 * E)
    ef_t = edge_feats.T.astype(bf)
    ea_t = edge_attrs.reshape(1, E)
    r1, r2, r3, r4 = (R1.astype(bf), R2.astype(bf), R3.astype(bf),
                      R4.astype(bf))
    x = _x_up(node_feats, W_up)
    msgs = []
    for s in range(S):
        w_s = _edge_weights(ef_t, ea_t, r1, r2, r3, r4, s)
        parts = _sc_message(x, w_s, ei_flat, s * E_SL)
        msgs.append(parts[0])
        msgs.append(parts[1])
    return _readout(msgs, node_attrs, W_lin, W_skip)


# speedup vs baseline: 1.3482x; 1.0149x over previous
import dataclasses
import functools

import jax
import jax.numpy as jnp
import numpy as np
from jax import lax
from jax.experimental import pallas as pl
from jax.experimental.pallas import tpu as pltpu
from jax.experimental.pallas import tpu_sc as plsc

N = 10000
E = 320000
D = 128
A = 10
RB = 8
H = 64
AVG = 32.0

NC = 2
NS = 16
NW = NC * NS

S = 2
E_SL = E // S
EPW = E_SL // NW
C = 40
NCHUNK = EPW // C
NR = 624

EB = 6400


def _silu(v):
    return v * (0.5 + 0.5 * jnp.tanh(0.5 * v))


def _x_up(node_feats, W_up):
    nb = 2000

    def body(nf_ref, w_ref, o_ref):
        o_ref[...] = jnp.dot(nf_ref[...], w_ref[...],
                             preferred_element_type=jnp.float32)

    return pl.pallas_call(
        body,
        grid=(N // nb,),
        in_specs=[pl.BlockSpec((nb, D), lambda i: (i, 0)),
                  pl.BlockSpec((D, D), lambda i: (0, 0))],
        out_specs=pl.BlockSpec((nb, D), lambda i: (i, 0)),
        out_shape=jax.ShapeDtypeStruct((N, D), jnp.float32),
    )(node_feats, W_up)


def _edge_weights(ef_t, ea_t, R1, R2, R3, R4, s):
    bf = jnp.bfloat16
    blk0 = s * (E_SL // EB)
    dn_t = (((0,), (0,)), ((), ()))

    def body(ef_ref, ea_ref, r1, r2, r3, r4, o_ref):
        h = _silu(lax.dot_general(ef_ref[...], r1[...], dn_t,
                                  preferred_element_type=jnp.float32))
        h = _silu(jnp.dot(h.astype(bf), r2[...],
                          preferred_element_type=jnp.float32))
        h = _silu(jnp.dot(h.astype(bf), r3[...],
                          preferred_element_type=jnp.float32))
        ea_col = lax.dot_general(ea_ref[...], jnp.ones((1, 1), jnp.float32),
                                 dn_t, preferred_element_type=jnp.float32)
        o_ref[...] = jnp.dot(h.astype(bf), r4[...],
                             preferred_element_type=jnp.float32) * ea_col

    return pl.pallas_call(
        body,
        grid=(E_SL // EB,),
        in_specs=[pl.BlockSpec((RB, EB), lambda i: (0, i + blk0)),
                  pl.BlockSpec((1, EB), lambda i: (0, i + blk0)),
                  pl.BlockSpec((RB, H), lambda i: (0, 0)),
                  pl.BlockSpec((H, H), lambda i: (0, 0)),
                  pl.BlockSpec((H, H), lambda i: (0, 0)),
                  pl.BlockSpec((H, D), lambda i: (0, 0))],
        out_specs=pl.BlockSpec((EB, D), lambda i: (i, 0)),
        out_shape=jax.ShapeDtypeStruct((E_SL, D), jnp.float32),
    )(ef_t, ea_t, R1, R2, R3, R4)


def _sc_message(x, w, ei_flat, eoff0):
    mesh = plsc.VectorSubcoreMesh(core_axis_name="c", subcore_axis_name="s")

    @functools.partial(
        pl.kernel,
        out_type=jax.ShapeDtypeStruct((NC, N, D), jnp.float32),
        mesh=mesh,
        scratch_types=[
            pltpu.VMEM((4, C), jnp.int32),
            pltpu.VMEM((4, C), jnp.int32),
            pltpu.VMEM((2, C, D), jnp.float32),
            pltpu.VMEM((2, C, D), jnp.float32),
            pltpu.VMEM_SHARED((N, D), jnp.float32),
            pltpu.SemaphoreType.DMA((4,)),
            pltpu.SemaphoreType.DMA((2,)),
            pltpu.SemaphoreType.DMA((2,)),
            pltpu.SemaphoreType.DMA((2,)),
        ],
    )
    def k(x_hbm, w_hbm, ei_hbm, out_hbm,
          si_v, ri_v, xs_v, wv_v, acc_sh, sem_i, sem_g, sem_w, sem_s):
        cid = lax.axis_index("c")
        sid = lax.axis_index("s")
        base = eoff0 + (cid * NS + sid) * EPW
        wbase = (cid * NS + sid) * EPW

        def issue_idx(g, slot):
            eoff = base + g * C
            pltpu.async_copy(ei_hbm.at[pl.ds(eoff, C)], si_v.at[slot],
                             sem_i.at[slot])
            pltpu.async_copy(ei_hbm.at[pl.ds(E + eoff, C)], ri_v.at[slot],
                             sem_i.at[slot])

        def wait_idx(g, slot):
            eoff = base + g * C
            pltpu.make_async_copy(ei_hbm.at[pl.ds(eoff, C)], si_v.at[slot],
                                  sem_i.at[slot]).wait()
            pltpu.make_async_copy(ei_hbm.at[pl.ds(E + eoff, C)],
                                  ri_v.at[slot], sem_i.at[slot]).wait()

        def issue_gather(slot, p):
            pltpu.async_copy(x_hbm.at[si_v.at[slot]], xs_v.at[p], sem_g.at[p])

        def wait_gather(slot, p):
            pltpu.make_async_copy(x_hbm.at[si_v.at[slot]], xs_v.at[p],
                                  sem_g.at[p]).wait()

        def issue_w(g, p):
            eoff = wbase + g * C
            pltpu.async_copy(w_hbm.at[pl.ds(eoff, C)], wv_v.at[p],
                             sem_w.at[p])

        def wait_w(g, p):
            eoff = wbase + g * C
            pltpu.make_async_copy(w_hbm.at[pl.ds(eoff, C)], wv_v.at[p],
                                  sem_w.at[p]).wait()

        def issue_scatter(p, slot):
            pltpu.async_copy(xs_v.at[p], acc_sh.at[ri_v.at[slot]],
                             sem_s.at[p], add=True)

        def wait_scatter(p, slot):
            pltpu.make_async_copy(xs_v.at[p], acc_sh.at[ri_v.at[slot]],
                                  sem_s.at[p]).wait()

        def multiply(p):
            xs_p = xs_v.at[p]
            w_p = wv_v.at[p]

            @pl.loop(0, C)
            def _(i):
                for j in range(D // 16):
                    sl = (pl.ds(i, 1), pl.ds(j * 16, 16))
                    xs_p.at[sl][...] = xs_p.at[sl][...] * w_p.at[sl][...]

        @pl.loop(0, C)
        def _(i):
            for j in range(D // 16):
                xs_v.at[0].at[pl.ds(i, 1), pl.ds(j * 16, 16)][...] = (
                    jnp.zeros((1, 16), jnp.float32))

        row0 = sid * NR
        zchunks = [C] * (NR // C)
        if NR % C:
            zchunks.append(NR % C)
        off = 0
        for cnt in zchunks:
            pltpu.sync_copy(xs_v.at[0].at[pl.ds(0, cnt)],
                            acc_sh.at[pl.ds(row0 + off, cnt)])
            off += cnt

        @pl.when(sid == NS - 1)
        def _():
            pltpu.sync_copy(xs_v.at[0].at[pl.ds(0, N - NS * NR)],
                            acc_sh.at[pl.ds(NS * NR, N - NS * NR)])

        plsc.subcore_barrier()

        issue_idx(0, 0)
        issue_idx(1, 1)
        wait_idx(0, 0)
        issue_gather(0, 0)
        issue_w(0, 0)

        @pl.loop(0, (NCHUNK - 1) // 4)
        def _(t):
            for b in range(4):
                g = t * 4 + b
                s1 = (b + 1) % 4
                p = b % 2
                q = (b + 1) % 2
                wait_idx(g + 1, s1)

                @pl.when(g >= 1)
                def _():
                    wait_scatter(q, (b + 3) % 4)

                issue_gather(s1, q)
                issue_w(g + 1, q)

                @pl.when(g + 2 <= NCHUNK - 1)
                def _():
                    issue_idx(g + 2, (b + 2) % 4)

                wait_gather(b % 4, p)
                wait_w(g, p)
                multiply(p)
                issue_scatter(p, b % 4)

        wait_scatter(1, 3)
        wait_gather(0, 0)
        wait_w(NCHUNK - 1, 0)
        multiply(0)
        pltpu.sync_copy(xs_v.at[0], acc_sh.at[ri_v.at[0]], add=True)

        plsc.subcore_barrier()
        pltpu.sync_copy(acc_sh.at[pl.ds(row0, NR)],
                        out_hbm.at[cid, pl.ds(row0, NR)])

        @pl.when(sid == NS - 1)
        def _():
            pltpu.sync_copy(acc_sh.at[pl.ds(NS * NR, N - NS * NR)],
                            out_hbm.at[cid, pl.ds(NS * NR, N - NS * NR)])

    return k(x, w, ei_flat)


def _readout(msgs, node_attrs, W_lin, W_skip):
    nb = 2000
    nm = len(msgs)

    def body(*refs):
        msg_refs = refs[:nm]
        na_ref, wl_ref, ws_ref, o_ref = refs[nm:]
        m = msg_refs[0][...]
        for r in msg_refs[1:]:
            m = m + r[...]
        m2 = jnp.dot(m, wl_ref[...],
                     preferred_element_type=jnp.float32) * (1.0 / AVG)
        na = na_ref[...]
        ws = ws_ref[...]
        acc = jnp.dot(m2 * na[:, 0:1], ws[:, 0, :],
                      preferred_element_type=jnp.float32)
        for a in range(1, A):
            acc = acc + jnp.dot(m2 * na[:, a:a + 1], ws[:, a, :],
                                preferred_element_type=jnp.float32)
        o_ref[...] = acc

    return pl.pallas_call(
        body,
        grid=(N // nb,),
        in_specs=[pl.BlockSpec((nb, D), lambda i: (i, 0))] * nm + [
            pl.BlockSpec((nb, A), lambda i: (i, 0)),
            pl.BlockSpec((D, D), lambda i: (0, 0)),
            pl.BlockSpec((D, A, D), lambda i: (0, 0, 0))],
        out_specs=pl.BlockSpec((nb, D), lambda i: (i, 0)),
        out_shape=jax.ShapeDtypeStruct((N, D), jnp.float32),
    )(*msgs, node_attrs, W_lin, W_skip)


def kernel(node_attrs, node_feats, edge_attrs, edge_feats, edge_index,
           W_up, R1, R2, R3, R4, W_lin, W_skip):
    bf = jnp.bfloat16
    ei_flat = edge_index.reshape(2 * E)
    ef_t = edge_feats.T.astype(bf)
    ea_t = edge_attrs.reshape(1, E)
    r1, r2, r3, r4 = (R1.astype(bf), R2.astype(bf), R3.astype(bf),
                      R4.astype(bf))
    x = _x_up(node_feats, W_up)
    msgs = []
    for s in range(S):
        w_s = _edge_weights(ef_t, ea_t, r1, r2, r3, r4, s)
        parts = _sc_message(x, w_s, ei_flat, s * E_SL)
        msgs.append(parts[0])
        msgs.append(parts[1])
    return _readout(msgs, node_attrs, W_lin, W_skip)
